# baseline (speedup 1.0000x reference)
import functools

import jax
import jax.numpy as jnp
from jax.experimental import pallas as pl
from jax.experimental.pallas import tpu as pltpu

_SMOOTHING = 0.1
_CONFIDENCE = 1.0 - _SMOOTHING
_IGNORE_INDEX = -100


def _half_losses(num_classes, block_r, pred_ref, tgt_half, tgt_base,
                 tgt_sm, g_ref):
    bulk = (num_classes // 128) * 128
    lane128 = jax.lax.broadcasted_iota(jnp.int32, (1, 128), 1)

    xb = pred_ref[:, :bulk]
    m = jnp.max(xb, axis=1, keepdims=True)
    sx = jnp.sum(xb, axis=1, keepdims=True)
    xt = pred_ref[:, bulk:num_classes] if bulk != num_classes else None
    if xt is not None:
        m = jnp.maximum(m, jnp.max(xt, axis=1, keepdims=True))
        sx = sx + jnp.sum(xt, axis=1, keepdims=True)
    s = jnp.sum(jnp.exp(xb - m), axis=1, keepdims=True)
    if xt is not None:
        s = s + jnp.sum(jnp.exp(xt - m), axis=1, keepdims=True)

    for r in range(block_r):
        t = tgt_sm[tgt_base + r]
        t = jnp.maximum(t, 0)
        al = jnp.minimum(t >> 7, bulk // 128 - 1) * 128
        chunk = pred_ref[pl.ds(r, 1), pl.ds(al, 128)]
        g_ref[pl.ds(r, 1), :] = jnp.where(lane128 == (t - al), chunk, 0.0)
    g = jnp.sum(g_ref[...], axis=1, keepdims=True)
    if xt is not None:
        lane_t = jax.lax.broadcasted_iota(
            jnp.int32, (1, num_classes - bulk), 1)
        g = g + jnp.sum(
            jnp.where(lane_t == (tgt_half - bulk), xt, 0.0),
            axis=1, keepdims=True)

    lse = m + jnp.log(s)
    sum_logp = sx - num_classes * lse
    logp_t = g - lse
    eps = _SMOOTHING / (num_classes - 1)
    row_loss = -eps * sum_logp - (_CONFIDENCE - eps) * logp_t
    maskf = (tgt_half != _IGNORE_INDEX).astype(jnp.float32)
    return row_loss * maskf, maskf


def _row_body(num_classes, block_r,
              tgt_sm, pa_ref, pb_ref, tgt_ref, rl_ref, mk_ref, ga_ref, gb_ref):
    j = pl.program_id(0)
    ta = tgt_ref[:block_r, :]
    tb = tgt_ref[block_r:, :]
    rla, mka = _half_losses(num_classes, block_r, pa_ref, ta,
                            j * 2 * block_r, tgt_sm, ga_ref)
    rlb, mkb = _half_losses(num_classes, block_r, pb_ref, tb,
                            j * 2 * block_r + block_r, tgt_sm, gb_ref)
    rl_ref[...] = jnp.concatenate([rla, rlb], axis=0)
    mk_ref[...] = jnp.concatenate([mka, mkb], axis=0)


def _mean_body(rl_ref, mk_ref, out_ref):
    out_ref[...] = (jnp.sum(rl_ref[...]) / jnp.sum(mk_ref[...])).reshape(1, 1)


def kernel(pred, target):
    n, num_classes = pred.shape
    block_r = 8
    nblocks = n // (2 * block_r)
    tgt2 = target.reshape(n, 1)

    rl, mk = pl.pallas_call(
        functools.partial(_row_body, num_classes, block_r),
        grid=(nblocks,),
        in_specs=[
            pl.BlockSpec(memory_space=pltpu.SMEM),
            pl.BlockSpec((block_r, num_classes), lambda j: (2 * j, 0)),
            pl.BlockSpec((block_r, num_classes), lambda j: (2 * j + 1, 0)),
            pl.BlockSpec((2 * block_r, 1), lambda j: (j, 0)),
        ],
        out_specs=[pl.BlockSpec((2 * block_r, 1), lambda j: (j, 0))] * 2,
        out_shape=[jax.ShapeDtypeStruct((n, 1), jnp.float32)] * 2,
        scratch_shapes=[pltpu.VMEM((block_r, 128), jnp.float32)] * 2,
        compiler_params=pltpu.CompilerParams(
            dimension_semantics=("parallel",)),
    )(target, pred, pred, tgt2)

    out = pl.pallas_call(
        _mean_body,
        out_shape=jax.ShapeDtypeStruct((1, 1), jnp.float32),
    )(rl, mk)
    return out[0, 0]


# speedup vs baseline: 1.1157x; 1.1157x over previous
import functools

import jax
import jax.numpy as jnp
from jax.experimental import pallas as pl
from jax.experimental.pallas import tpu as pltpu

_SMOOTHING = 0.1
_CONFIDENCE = 1.0 - _SMOOTHING
_IGNORE_INDEX = -100
_NSTREAM = 4


def _half_losses(num_classes, block_r, pred_ref, tgt_half, tgt_base,
                 tgt_sm, g_ref):
    bulk = (num_classes // 128) * 128
    lane128 = jax.lax.broadcasted_iota(jnp.int32, (1, 128), 1)

    xb = pred_ref[:, :bulk]
    m = jnp.max(xb, axis=1, keepdims=True)
    sx = jnp.sum(xb, axis=1, keepdims=True)
    xt = pred_ref[:, bulk:num_classes] if bulk != num_classes else None
    if xt is not None:
        m = jnp.maximum(m, jnp.max(xt, axis=1, keepdims=True))
        sx = sx + jnp.sum(xt, axis=1, keepdims=True)
    s = jnp.sum(jnp.exp(xb - m), axis=1, keepdims=True)
    if xt is not None:
        s = s + jnp.sum(jnp.exp(xt - m), axis=1, keepdims=True)

    for r in range(block_r):
        t = tgt_sm[tgt_base + r]
        t = jnp.maximum(t, 0)
        al = jnp.minimum(t >> 7, bulk // 128 - 1) * 128
        chunk = pred_ref[pl.ds(r, 1), pl.ds(al, 128)]
        g_ref[pl.ds(r, 1), :] = jnp.where(lane128 == (t - al), chunk, 0.0)
    g = jnp.sum(g_ref[...], axis=1, keepdims=True)
    if xt is not None:
        lane_t = jax.lax.broadcasted_iota(
            jnp.int32, (1, num_classes - bulk), 1)
        g = g + jnp.sum(
            jnp.where(lane_t == (tgt_half - bulk), xt, 0.0),
            axis=1, keepdims=True)

    lse = m + jnp.log(s)
    sum_logp = sx - num_classes * lse
    logp_t = g - lse
    eps = _SMOOTHING / (num_classes - 1)
    row_loss = -eps * sum_logp - (_CONFIDENCE - eps) * logp_t
    maskf = (tgt_half != _IGNORE_INDEX).astype(jnp.float32)
    return row_loss * maskf, maskf


def _row_body(num_classes, block_r, tgt_sm, *refs):
    j = pl.program_id(0)
    pred_refs = refs[:_NSTREAM]
    tgt_ref = refs[_NSTREAM]
    rl_ref, mk_ref = refs[_NSTREAM + 1], refs[_NSTREAM + 2]
    g_refs = refs[_NSTREAM + 3:]
    rls, mks = [], []
    for h in range(_NSTREAM):
        th = tgt_ref[h * block_r:(h + 1) * block_r, :]
        rl, mk = _half_losses(num_classes, block_r, pred_refs[h], th,
                              j * _NSTREAM * block_r + h * block_r,
                              tgt_sm, g_refs[h])
        rls.append(rl)
        mks.append(mk)
    rl_ref[...] = jnp.concatenate(rls, axis=0)
    mk_ref[...] = jnp.concatenate(mks, axis=0)


def _mean_body(rl_ref, mk_ref, out_ref):
    out_ref[...] = (jnp.sum(rl_ref[...]) / jnp.sum(mk_ref[...])).reshape(1, 1)


def kernel(pred, target):
    n, num_classes = pred.shape
    block_r = 16
    rows_per_step = _NSTREAM * block_r
    nblocks = n // rows_per_step
    tgt2 = target.reshape(n, 1)

    def pspec(h):
        return pl.BlockSpec((block_r, num_classes),
                            lambda j, h=h: (_NSTREAM * j + h, 0))

    rl, mk = pl.pallas_call(
        functools.partial(_row_body, num_classes, block_r),
        grid=(nblocks,),
        in_specs=[pl.BlockSpec(memory_space=pltpu.SMEM)]
        + [pspec(h) for h in range(_NSTREAM)]
        + [pl.BlockSpec((rows_per_step, 1), lambda j: (j, 0))],
        out_specs=[pl.BlockSpec((rows_per_step, 1), lambda j: (j, 0))] * 2,
        out_shape=[jax.ShapeDtypeStruct((n, 1), jnp.float32)] * 2,
        scratch_shapes=[pltpu.VMEM((block_r, 128), jnp.float32)] * _NSTREAM,
        compiler_params=pltpu.CompilerParams(
            dimension_semantics=("parallel",)),
    )(target, *([pred] * _NSTREAM), tgt2)

    out = pl.pallas_call(
        _mean_body,
        out_shape=jax.ShapeDtypeStruct((1, 1), jnp.float32),
    )(rl, mk)
    return out[0, 0]
